# Initial kernel scaffold; baseline (speedup 1.0000x reference)
import jax
import jax.numpy as jnp
from jax.experimental import pallas as pl


def kernel(x, y, visible, memory):
    raise NotImplementedError("write your pallas kernel here")


# speedup vs baseline: 1.5401x; 1.5401x over previous
import jax
import jax.numpy as jnp
from jax.experimental import pallas as pl

_B, _NPOS, _NNEG, _D, _NLEM = 8, 128, 64, 128, 16384
_MOM = 0.5
_T = 2048
_GRID = _NLEM // _T


def _renorm(m):
    s = jnp.sum(m * m, axis=1, keepdims=True)
    return m / jnp.maximum(jnp.sqrt(s), 1e-12)


def _body(x3_ref, xneg_ref, vis_ref, mem_ref,
          sim_ref, noise_ref, newmem_ref, acc_ref):
    i = pl.program_id(0)
    x3 = x3_ref[...]
    xf = x3.reshape(_B * _NPOS, _D)
    mem = mem_ref[...]
    sim_ref[...] = jax.lax.dot_general(
        xf, mem, (((1,), (1,)), ((), ())), preferred_element_type=jnp.float32)

    @pl.when(i == 0)
    def _():
        vis = vis_ref[...]
        xneg = xneg_ref[...]
        mem_head = mem[0:_NPOS, :]
        noise_ref[...] = jax.lax.dot_general(
            xneg, mem_head, (((1,), (1,)), ((), ())),
            preferred_element_type=jnp.float32)
        get = jnp.mean(x3 * vis[..., None], axis=0)
        head = mem_head * _MOM + get * (1.0 - _MOM)
        newmem_ref[0:_NPOS, :] = _renorm(head)
        newmem_ref[_NPOS:_NPOS + _B * _NNEG, :] = _renorm(xneg)
        newmem_ref[_NPOS + _B * _NNEG:, :] = _renorm(mem[_NPOS + _B * _NNEG:, :])
        acc_ref[...] = jnp.sum((vis > 0).astype(jnp.int32), axis=0,
                               keepdims=True)

    @pl.when(i != 0)
    def _():
        newmem_ref[...] = _renorm(mem)


def kernel(x, y, visible, memory):
    x3 = x[:, 0:_NPOS, :]
    xneg = x[:, _NPOS:, :].reshape(_B * _NNEG, _D)

    sim, noise, new_memory, acc = pl.pallas_call(
        _body,
        grid=(_GRID,),
        in_specs=[
            pl.BlockSpec((_B, _NPOS, _D), lambda i: (0, 0, 0)),
            pl.BlockSpec((_B * _NNEG, _D), lambda i: (0, 0)),
            pl.BlockSpec((_B, _NPOS), lambda i: (0, 0)),
            pl.BlockSpec((_T, _D), lambda i: (i, 0)),
        ],
        out_specs=[
            pl.BlockSpec((_B * _NPOS, _T), lambda i: (0, i)),
            pl.BlockSpec((_B * _NNEG, _NPOS), lambda i: (0, 0)),
            pl.BlockSpec((_T, _D), lambda i: (i, 0)),
            pl.BlockSpec((1, _NPOS), lambda i: (0, 0)),
        ],
        out_shape=[
            jax.ShapeDtypeStruct((_B * _NPOS, _NLEM), jnp.float32),
            jax.ShapeDtypeStruct((_B * _NNEG, _NPOS), jnp.float32),
            jax.ShapeDtypeStruct((_NLEM, _D), jnp.float32),
            jax.ShapeDtypeStruct((1, _NPOS), jnp.int32),
        ],
    )(x3, xneg, visible, memory)

    similarity = sim.reshape(_B, _NPOS, _NLEM)
    noise_similarity = noise.reshape(_B, _NNEG, _NPOS)
    y_idx = y.astype(jnp.int32)
    accumulate_delta = acc.reshape(_NPOS)
    return (similarity, y_idx, noise_similarity, new_memory, accumulate_delta)
